# baseline (speedup 1.0000x reference)
import functools

import jax
import jax.numpy as jnp
from jax import lax
from jax.experimental import pallas as pl
from jax.experimental.pallas import tpu as pltpu
from jax.experimental.pallas import tpu_sc as plsc

N = 10000
D_IN = 128
DK = 16
E = 160000

NC = 2
NS = 16
NW = NC * NS
EPW = E // NW
CH = 125
NCH = EPW // CH
ZCH = 128
RPT = 640
NPAD = NS * RPT

def _fill(ref, rows, val):
    def body(i, _):
        ref[i] = jnp.full((DK,), val, jnp.float32)
        return 0
    lax.fori_loop(0, rows, body, 0)


def _zero_stripe(zbuf_v, acc_sh, s):
    _fill(zbuf_v, ZCH, 0.0)
    for k in range(RPT // ZCH):
        pltpu.sync_copy(zbuf_v, acc_sh.at[pl.ds(s * RPT + k * ZCH, ZCH)])


def _writeback(acc_sh, out_hbm, c, s):
    pltpu.sync_copy(acc_sh.at[pl.ds(s * RPT, RPT)],
                    out_hbm.at[c, pl.ds(s * RPT, RPT)])


@functools.cache
def _sc_kernels():
    mesh = plsc.VectorSubcoreMesh(core_axis_name="c", subcore_axis_name="s")

    @functools.partial(
        pl.kernel,
        out_type=jax.ShapeDtypeStruct((NC, NPAD, DK), jnp.float32),
        mesh=mesh,
        compiler_params=pltpu.CompilerParams(use_tc_tiling_on_sc=False),
        scratch_types=[
            pltpu.VMEM((NCH, CH), jnp.int32),
            pltpu.VMEM((CH, DK), jnp.float32),
            pltpu.VMEM((ZCH, DK), jnp.float32),
            pltpu.VMEM_SHARED((NPAD, DK), jnp.float32),
        ],
    )
    def _sc_deg(dst_hbm, out_hbm, idx_v, buf_v, zbuf_v, acc_sh):
        c = lax.axis_index("c")
        s = lax.axis_index("s")
        w = c * NS + s
        pltpu.sync_copy(dst_hbm.at[w], idx_v)
        _zero_stripe(zbuf_v, acc_sh, s)
        plsc.subcore_barrier()
        _fill(buf_v, CH, 1.0)

        def chunk(j, _):
            pltpu.sync_copy(buf_v, acc_sh.at[idx_v.at[j]], add=True)
            return 0
        lax.fori_loop(0, NCH, chunk, 0)
        plsc.subcore_barrier()
        _writeback(acc_sh, out_hbm, c, s)

    @functools.partial(
        pl.kernel,
        out_type=jax.ShapeDtypeStruct((NC, NPAD, DK), jnp.float32),
        mesh=mesh,
        compiler_params=pltpu.CompilerParams(use_tc_tiling_on_sc=False),
        scratch_types=[
            pltpu.VMEM((NCH, CH), jnp.int32),
            pltpu.VMEM((NCH, CH), jnp.int32),
            pltpu.VMEM((CH, DK), jnp.float32),
            pltpu.VMEM((CH, DK), jnp.float32),
            pltpu.VMEM((ZCH, DK), jnp.float32),
            pltpu.VMEM_SHARED((NPAD, DK), jnp.float32),
            pltpu.SemaphoreType.DMA,
            pltpu.SemaphoreType.DMA,
        ],
    )
    def _sc_agg(g_hbm, src_hbm, dst_hbm, out_hbm, src_v, dst_v, buf0_v, buf1_v,
                zbuf_v, acc_sh, sem0, sem1):
        c = lax.axis_index("c")
        s = lax.axis_index("s")
        w = c * NS + s
        pltpu.sync_copy(src_hbm.at[w], src_v)
        pltpu.sync_copy(dst_hbm.at[w], dst_v)
        _zero_stripe(zbuf_v, acc_sh, s)
        plsc.subcore_barrier()

        pltpu.async_copy(g_hbm.at[src_v.at[0]], buf0_v, sem0)

        def pair(p, _):
            j0 = 2 * p
            pltpu.async_copy(g_hbm.at[src_v.at[j0 + 1]], buf1_v, sem1)
            pltpu.make_async_copy(g_hbm.at[src_v.at[j0]], buf0_v, sem0).wait()
            pltpu.sync_copy(buf0_v, acc_sh.at[dst_v.at[j0]], add=True)

            @pl.when(j0 + 2 < NCH)
            def _():
                pltpu.async_copy(g_hbm.at[src_v.at[j0 + 2]], buf0_v, sem0)
            pltpu.make_async_copy(g_hbm.at[src_v.at[j0 + 1]], buf1_v,
                                  sem1).wait()
            pltpu.sync_copy(buf1_v, acc_sh.at[dst_v.at[j0 + 1]], add=True)
            return 0
        lax.fori_loop(0, NCH // 2, pair, 0)
        plsc.subcore_barrier()
        _writeback(acc_sh, out_hbm, c, s)

    return _sc_deg, _sc_agg


def _tc_prep(x, W, deg0, deg1):
    def body(x_ref, w_ref, d0_ref, d1_ref, g_ref, dinv_ref):
        h = lax.dot_general(x_ref[...], w_ref[...], (((0,), (0,)), ((), ())),
                            preferred_element_type=jnp.float32)
        dinv = lax.rsqrt(d0_ref[...] + d1_ref[...] + 1.0)
        dinv_ref[...] = dinv
        g_ref[...] = h * dinv
    return pl.pallas_call(
        body,
        out_shape=(jax.ShapeDtypeStruct((N, DK), jnp.float32),
                   jax.ShapeDtypeStruct((N, DK), jnp.float32)),
    )(x, W, deg0, deg1)


def _tc_z(a0, a1, g, dinv, b2):
    def body(a0_ref, a1_ref, g_ref, dinv_ref, b_ref, z_ref, zt_ref):
        z = (dinv_ref[...] * (a0_ref[...] + a1_ref[...] + g_ref[...])
             + b_ref[...])
        z_ref[...] = z
        zt_ref[...] = z.T
    return pl.pallas_call(
        body,
        out_shape=(jax.ShapeDtypeStruct((N, DK), jnp.float32),
                   jax.ShapeDtypeStruct((DK, N), jnp.float32)),
    )(a0, a1, g, dinv, b2)


BM = 512


def _tc_decode(z, zt):
    def body(z_ref, zt_ref, o_ref):
        t = jnp.dot(z_ref[...], zt_ref[...], preferred_element_type=jnp.float32)
        o_ref[...] = 1.0 / (1.0 + jnp.exp(-t))
    return pl.pallas_call(
        body,
        grid=(pl.cdiv(N, BM),),
        in_specs=[pl.BlockSpec((BM, DK), lambda i: (i, 0)),
                  pl.BlockSpec((DK, N), lambda i: (0, 0))],
        out_specs=pl.BlockSpec((BM, N), lambda i: (i, 0)),
        out_shape=jax.ShapeDtypeStruct((N, N), jnp.float32),
    )(z, zt)


def kernel(x, edge_index, W_enc, b_enc):
    src3 = edge_index[0].reshape(NW, NCH, CH)
    dst3 = edge_index[1].reshape(NW, NCH, CH)
    sc_deg, sc_agg = _sc_kernels()
    degp = sc_deg(dst3)
    g, dinv = _tc_prep(x, W_enc, degp[0, :N], degp[1, :N])
    accp = sc_agg(g, src3, dst3)
    z, zt = _tc_z(accp[0, :N], accp[1, :N], g, dinv, b_enc.reshape(1, DK))
    return _tc_decode(z, zt)


# speedup vs baseline: 1.0431x; 1.0431x over previous
import functools

import jax
import jax.numpy as jnp
from jax import lax
from jax.experimental import pallas as pl
from jax.experimental.pallas import tpu as pltpu
from jax.experimental.pallas import tpu_sc as plsc

N = 10000
D_IN = 128
DK = 16
E = 160000

NC = 2
NS = 16
NW = NC * NS
EPW = E // NW
CH = 125
NCH = EPW // CH
ZCH = 128
RPT = 640
NPAD = NS * RPT

def _fill(ref, rows, val):
    def body(i, _):
        ref[i] = jnp.full((DK,), val, jnp.float32)
        return 0
    lax.fori_loop(0, rows, body, 0)


def _zero_stripe(zbuf_v, acc_sh, s):
    _fill(zbuf_v, ZCH, 0.0)
    for k in range(RPT // ZCH):
        pltpu.sync_copy(zbuf_v, acc_sh.at[pl.ds(s * RPT + k * ZCH, ZCH)])


LAST = N - (NS - 1) * RPT


def _writeback(acc_sh, out0_hbm, out1_hbm, c, s):
    for core, out in ((0, out0_hbm), (1, out1_hbm)):
        @pl.when(jnp.logical_and(c == core, s < NS - 1))
        def _():
            pltpu.sync_copy(acc_sh.at[pl.ds(s * RPT, RPT)],
                            out.at[pl.ds(s * RPT, RPT)])

        @pl.when(jnp.logical_and(c == core, s == NS - 1))
        def _():
            pltpu.sync_copy(acc_sh.at[pl.ds((NS - 1) * RPT, LAST)],
                            out.at[pl.ds((NS - 1) * RPT, LAST)])


@functools.cache
def _sc_kernels():
    mesh = plsc.VectorSubcoreMesh(core_axis_name="c", subcore_axis_name="s")

    out2 = (jax.ShapeDtypeStruct((N, DK), jnp.float32),
            jax.ShapeDtypeStruct((N, DK), jnp.float32))

    @functools.partial(
        pl.kernel,
        out_type=out2,
        mesh=mesh,
        compiler_params=pltpu.CompilerParams(use_tc_tiling_on_sc=False),
        scratch_types=[
            pltpu.VMEM((NCH, CH), jnp.int32),
            pltpu.VMEM((CH, DK), jnp.float32),
            pltpu.VMEM((ZCH, DK), jnp.float32),
            pltpu.VMEM_SHARED((NPAD, DK), jnp.float32),
            pltpu.SemaphoreType.DMA,
            pltpu.SemaphoreType.DMA,
        ],
    )
    def _sc_deg(dst_hbm, out0_hbm, out1_hbm, idx_v, buf_v, zbuf_v, acc_sh,
                sem0, sem1):
        c = lax.axis_index("c")
        s = lax.axis_index("s")
        w = c * NS + s
        pltpu.sync_copy(dst_hbm.at[w], idx_v)
        _zero_stripe(zbuf_v, acc_sh, s)
        plsc.subcore_barrier()
        _fill(buf_v, CH, 1.0)

        pltpu.async_copy(buf_v, acc_sh.at[idx_v.at[0]], sem0, add=True)

        def pair(p, _):
            j0 = 2 * p
            pltpu.async_copy(buf_v, acc_sh.at[idx_v.at[j0 + 1]], sem1,
                             add=True)
            pltpu.make_async_copy(buf_v, acc_sh.at[idx_v.at[j0]], sem0).wait()

            @pl.when(j0 + 2 < NCH)
            def _():
                pltpu.async_copy(buf_v, acc_sh.at[idx_v.at[j0 + 2]], sem0,
                                 add=True)
            pltpu.make_async_copy(buf_v, acc_sh.at[idx_v.at[j0 + 1]],
                                  sem1).wait()
            return 0
        lax.fori_loop(0, NCH // 2, pair, 0)
        plsc.subcore_barrier()
        _writeback(acc_sh, out0_hbm, out1_hbm, c, s)

    @functools.partial(
        pl.kernel,
        out_type=out2,
        mesh=mesh,
        compiler_params=pltpu.CompilerParams(use_tc_tiling_on_sc=False),
        scratch_types=[
            pltpu.VMEM((NCH, CH), jnp.int32),
            pltpu.VMEM((NCH, CH), jnp.int32),
            pltpu.VMEM((CH, DK), jnp.float32),
            pltpu.VMEM((CH, DK), jnp.float32),
            pltpu.VMEM((ZCH, DK), jnp.float32),
            pltpu.VMEM_SHARED((NPAD, DK), jnp.float32),
            pltpu.SemaphoreType.DMA,
            pltpu.SemaphoreType.DMA,
        ],
    )
    def _sc_agg(g_hbm, src_hbm, dst_hbm, out0_hbm, out1_hbm, src_v, dst_v,
                buf0_v, buf1_v, zbuf_v, acc_sh, sem0, sem1):
        c = lax.axis_index("c")
        s = lax.axis_index("s")
        w = c * NS + s
        pltpu.sync_copy(src_hbm.at[w], src_v)
        pltpu.sync_copy(dst_hbm.at[w], dst_v)
        _zero_stripe(zbuf_v, acc_sh, s)
        plsc.subcore_barrier()

        pltpu.async_copy(g_hbm.at[src_v.at[0]], buf0_v, sem0)

        def pair(p, _):
            j0 = 2 * p
            pltpu.async_copy(g_hbm.at[src_v.at[j0 + 1]], buf1_v, sem1)
            pltpu.make_async_copy(g_hbm.at[src_v.at[j0]], buf0_v, sem0).wait()
            pltpu.sync_copy(buf0_v, acc_sh.at[dst_v.at[j0]], add=True)

            @pl.when(j0 + 2 < NCH)
            def _():
                pltpu.async_copy(g_hbm.at[src_v.at[j0 + 2]], buf0_v, sem0)
            pltpu.make_async_copy(g_hbm.at[src_v.at[j0 + 1]], buf1_v,
                                  sem1).wait()
            pltpu.sync_copy(buf1_v, acc_sh.at[dst_v.at[j0 + 1]], add=True)
            return 0
        lax.fori_loop(0, NCH // 2, pair, 0)
        plsc.subcore_barrier()
        _writeback(acc_sh, out0_hbm, out1_hbm, c, s)

    return _sc_deg, _sc_agg


def _tc_prep(x, W, deg0, deg1):
    def body(x_ref, w_ref, d0_ref, d1_ref, g_ref, dinv_ref):
        h = lax.dot_general(x_ref[...], w_ref[...], (((0,), (0,)), ((), ())),
                            preferred_element_type=jnp.float32)
        dinv = lax.rsqrt(d0_ref[...] + d1_ref[...] + 1.0)
        dinv_ref[...] = dinv
        g_ref[...] = h * dinv
    return pl.pallas_call(
        body,
        out_shape=(jax.ShapeDtypeStruct((N, DK), jnp.float32),
                   jax.ShapeDtypeStruct((N, DK), jnp.float32)),
    )(x, W, deg0, deg1)


def _tc_z(a0, a1, g, dinv, b2):
    def body(a0_ref, a1_ref, g_ref, dinv_ref, b_ref, z_ref, zt_ref):
        z = (dinv_ref[...] * (a0_ref[...] + a1_ref[...] + g_ref[...])
             + b_ref[...])
        z_ref[...] = z
        zt_ref[...] = z.T
    return pl.pallas_call(
        body,
        out_shape=(jax.ShapeDtypeStruct((N, DK), jnp.float32),
                   jax.ShapeDtypeStruct((DK, N), jnp.float32)),
    )(a0, a1, g, dinv, b2)


BM = 256


def _tc_decode(z, zt):
    def body(z_ref, zt_ref, o_ref):
        t = jnp.dot(z_ref[...], zt_ref[...], preferred_element_type=jnp.float32)
        o_ref[...] = 1.0 / (1.0 + jnp.exp(-t))
    return pl.pallas_call(
        body,
        grid=(pl.cdiv(N, BM),),
        in_specs=[pl.BlockSpec((BM, DK), lambda i: (i, 0)),
                  pl.BlockSpec((DK, N), lambda i: (0, 0))],
        out_specs=pl.BlockSpec((BM, N), lambda i: (i, 0)),
        out_shape=jax.ShapeDtypeStruct((N, N), jnp.float32),
    )(z, zt)


def kernel(x, edge_index, W_enc, b_enc):
    src3 = edge_index[0].reshape(NW, NCH, CH)
    dst3 = edge_index[1].reshape(NW, NCH, CH)
    sc_deg, sc_agg = _sc_kernels()
    deg0, deg1 = sc_deg(dst3)
    g, dinv = _tc_prep(x, W_enc, deg0, deg1)
    acc0, acc1 = sc_agg(g, src3, dst3)
    z, zt = _tc_z(acc0, acc1, g, dinv, b_enc.reshape(1, DK))
    return _tc_decode(z, zt)
